# Initial kernel scaffold; baseline (speedup 1.0000x reference)
import jax
import jax.numpy as jnp
from jax import lax
from jax.experimental import pallas as pl
from jax.experimental.pallas import tpu as pltpu
from jax.experimental.pallas import tpu_sc as plsc

N = 10000
E = 320000
F = 128
H = 128
C = 10
G = 16

NC = 2
NS = 16
NW = NC * NS
EPW = E // NW
K = 80
NCHUNK = EPW // K
RPT = N // NS
ZR = 125

_f32 = jnp.float32


def _dot(a, b):
    return jnp.dot(a, b, preferred_element_type=_f32)


def _tc_fold1(x_ref, W2_ref, b2_ref, We_ref, be_ref, p_ref, q_ref):
    x = x_ref[...]
    We = We_ref[...]
    A = _dot(W2_ref[0:F, :], We)
    B = _dot(W2_ref[F:2 * F, :], We)
    c = _dot(b2_ref[...], We) + be_ref[...]
    p_ref[...] = x + _dot(x, A)
    q_ref[...] = _dot(x, B) + c


def _bn_mlp(u, Wa, ba, Wb, bb):
    t = _dot(u, Wa) + ba
    mu = jnp.mean(t, axis=0, keepdims=True)
    tc = t - mu
    var = jnp.mean(tc * tc, axis=0, keepdims=True)
    r = jnp.maximum(tc / jnp.sqrt(var + 1e-5), 0.0)
    return _dot(r, Wb) + bb


def _tc_mid(x_ref, pa_ref, W1a_ref, b1a_ref, W1b_ref, b1b_ref,
            W3_ref, b3_ref, We2_ref, be2_ref, p_ref, q_ref, xr_ref):
    u = x_ref[...] + pa_ref[0] + pa_ref[1]
    h = _bn_mlp(u, W1a_ref[...], b1a_ref[...], W1b_ref[...], b1b_ref[...])
    xr = jnp.maximum(h, 0.0)
    We2 = We2_ref[...]
    A = _dot(W3_ref[0:H, :], We2)
    B = _dot(W3_ref[H:2 * H, :], We2)
    c = _dot(b3_ref[...], We2) + be2_ref[...]
    p_ref[...] = xr + _dot(h, A)
    q_ref[...] = _dot(h, B) + c
    xr_ref[...] = xr


def _tc_final(xr_ref, pb_ref, W2a_ref, b2a_ref, W2b_ref, b2b_ref,
              batch_ref, Wl1_ref, bl1_ref, out_ref):
    u = xr_ref[...] + pb_ref[0] + pb_ref[1]
    h2 = _bn_mlp(u, W2a_ref[...], b2a_ref[...], W2b_ref[...], b2b_ref[...])
    hr = jnp.maximum(h2, 0.0)
    onehot = (batch_ref[...] ==
              lax.broadcasted_iota(jnp.int32, (N, G), 1)).astype(_f32)
    pooled = lax.dot_general(onehot, hr, (((0,), (0,)), ((), ())),
                             preferred_element_type=_f32)
    logits = _dot(pooled, Wl1_ref[...]) + bl1_ref[...]
    m = jnp.max(logits, axis=1, keepdims=True)
    lse = jnp.log(jnp.sum(jnp.exp(logits - m), axis=1, keepdims=True)) + m
    out_ref[...] = logits - lse


def _sc_edge_body(p_hbm, q_hbm, src_hbm, dst_hbm, out_hbm,
                  aggr_sh, sidx, didx, pbuf, qbuf, zbuf, sem_p, sem_q):
    c = lax.axis_index("c")
    s = lax.axis_index("s")

    @pl.loop(0, ZR)
    def _zrow(r):
        for j in range(8):
            zbuf[r, pl.ds(j * 16, 16)] = jnp.zeros((16,), _f32)

    for j in range(RPT // ZR):
        pltpu.sync_copy(zbuf, aggr_sh.at[pl.ds(s * RPT + j * ZR, ZR)])
    plsc.subcore_barrier()

    base = (c * NS + s) * EPW

    @pl.loop(0, NCHUNK)
    def _chunk(k):
        off = base + k * K
        pltpu.sync_copy(src_hbm.at[pl.ds(off, K)], sidx)
        pltpu.sync_copy(dst_hbm.at[pl.ds(off, K)], didx)
        cpp = pltpu.async_copy(p_hbm.at[sidx], pbuf, sem_p)
        cpq = pltpu.async_copy(q_hbm.at[didx], qbuf, sem_q)
        cpp.wait()
        cpq.wait()

        @pl.loop(0, K)
        def _row(r):
            for j in range(8):
                sl = pl.ds(j * 16, 16)
                pbuf[r, sl] = jnp.maximum(pbuf[r, sl] + qbuf[r, sl], 0.0)

        pltpu.sync_copy(pbuf, aggr_sh.at[didx], add=True)

    plsc.subcore_barrier()
    pltpu.sync_copy(aggr_sh.at[pl.ds(s * RPT, RPT)],
                    out_hbm.at[c, pl.ds(s * RPT, RPT)])


_sc_edge = pl.kernel(
    _sc_edge_body,
    out_type=jax.ShapeDtypeStruct((NC, N, F), _f32),
    mesh=plsc.VectorSubcoreMesh(core_axis_name="c", subcore_axis_name="s",
                                num_cores=NC, num_subcores=NS),
    scratch_types=[
        pltpu.VMEM_SHARED((N, F), _f32),
        pltpu.VMEM((K,), jnp.int32),
        pltpu.VMEM((K,), jnp.int32),
        pltpu.VMEM((K, F), _f32),
        pltpu.VMEM((K, F), _f32),
        pltpu.VMEM((ZR, F), _f32),
        pltpu.SemaphoreType.DMA,
        pltpu.SemaphoreType.DMA,
    ],
)


def kernel(x, edge_index, batch, W_lin2, b_lin2, We1, be1, W1a, b1a, W1b, b1b,
           W_lin3, b_lin3, We2, be2, W2a, b2a, W2b, b2b, W_lin1, b_lin1):
    src = edge_index[0]
    dst = edge_index[1]
    sds = jax.ShapeDtypeStruct

    p1, q1 = pl.pallas_call(
        _tc_fold1,
        out_shape=[sds((N, F), _f32), sds((N, F), _f32)],
    )(x, W_lin2, b_lin2.reshape(1, H), We1, be1.reshape(1, F))

    pa = _sc_edge(p1, q1, src, dst)

    p2, q2, xr = pl.pallas_call(
        _tc_mid,
        out_shape=[sds((N, H), _f32), sds((N, H), _f32), sds((N, H), _f32)],
    )(x, pa, W1a, b1a.reshape(1, H), W1b, b1b.reshape(1, H),
      W_lin3, b_lin3.reshape(1, H), We2, be2.reshape(1, H))

    pb = _sc_edge(p2, q2, src, dst)

    out = pl.pallas_call(
        _tc_final,
        out_shape=sds((G, C), _f32),
    )(xr, pb, W2a, b2a.reshape(1, H), W2b, b2b.reshape(1, H),
      batch.reshape(N, 1), W_lin1, b_lin1.reshape(1, C))

    return out


# speedup vs baseline: 5.5453x; 5.5453x over previous
import functools

import jax
import jax.numpy as jnp
from jax import lax
from jax.experimental import pallas as pl
from jax.experimental.pallas import tpu as pltpu
from jax.experimental.pallas import tpu_sc as plsc

N = 10000
E = 320000
F = 128
H = 128
C = 10
G = 16

NC = 2
NS = 16
NW = NC * NS
EPW = E // NW
K = 80
NCHUNK = EPW // K
RPT = 632
NP = NS * RPT
ZR = 8

_f32 = jnp.float32


def _dot(a, b):
    return jnp.dot(a, b, preferred_element_type=_f32)


def _tc_fold1(x_ref, W2_ref, b2_ref, We_ref, be_ref, p_ref, q_ref):
    x = x_ref[...]
    We = We_ref[...]
    A = _dot(W2_ref[0:F, :], We)
    B = _dot(W2_ref[F:2 * F, :], We)
    c = _dot(b2_ref[...], We) + be_ref[...]
    p_ref[...] = x + _dot(x, A)
    q_ref[...] = _dot(x, B) + c


def _bn_mlp(u, Wa, ba, Wb, bb):
    t = _dot(u, Wa) + ba
    mu = jnp.mean(t, axis=0, keepdims=True)
    tc = t - mu
    var = jnp.mean(tc * tc, axis=0, keepdims=True)
    r = jnp.maximum(tc / jnp.sqrt(var + 1e-5), 0.0)
    return _dot(r, Wb) + bb


def _tc_mid(x_ref, pa_ref, W1a_ref, b1a_ref, W1b_ref, b1b_ref,
            W3_ref, b3_ref, We2_ref, be2_ref, p_ref, q_ref, xr_ref):
    u = x_ref[...] + pa_ref[0, :N, :] + pa_ref[1, :N, :]
    h = _bn_mlp(u, W1a_ref[...], b1a_ref[...], W1b_ref[...], b1b_ref[...])
    xr = jnp.maximum(h, 0.0)
    We2 = We2_ref[...]
    A = _dot(W3_ref[0:H, :], We2)
    B = _dot(W3_ref[H:2 * H, :], We2)
    c = _dot(b3_ref[...], We2) + be2_ref[...]
    p_ref[...] = xr + _dot(h, A)
    q_ref[...] = _dot(h, B) + c
    xr_ref[...] = xr


def _tc_final(xr_ref, pb_ref, W2a_ref, b2a_ref, W2b_ref, b2b_ref,
              batch_ref, Wl1_ref, bl1_ref, out_ref):
    u = xr_ref[...] + pb_ref[0, :N, :] + pb_ref[1, :N, :]
    h2 = _bn_mlp(u, W2a_ref[...], b2a_ref[...], W2b_ref[...], b2b_ref[...])
    hr = jnp.maximum(h2, 0.0)
    onehot = (batch_ref[...] ==
              lax.broadcasted_iota(jnp.int32, (N, G), 1)).astype(_f32)
    pooled = lax.dot_general(onehot, hr, (((0,), (0,)), ((), ())),
                             preferred_element_type=_f32)
    logits = _dot(pooled, Wl1_ref[...]) + bl1_ref[...]
    m = jnp.max(logits, axis=1, keepdims=True)
    lse = jnp.log(jnp.sum(jnp.exp(logits - m), axis=1, keepdims=True)) + m
    out_ref[...] = logits - lse


def _sc_edge_body(p_hbm, q_hbm, src_hbm, dst_hbm, out_hbm,
                  aggr_sh, sidx, didx, pbuf, qbuf, zbuf, sem_p, sem_q):
    c = lax.axis_index("c")
    s = lax.axis_index("s")

    @pl.loop(0, ZR)
    def _zrow(r):
        for j in range(8):
            zbuf[r, pl.ds(j * 16, 16)] = jnp.zeros((16,), _f32)

    @pl.loop(0, RPT // ZR)
    def _zcopy(j):
        off = pl.multiple_of(s * RPT + j * ZR, 8)
        pltpu.sync_copy(zbuf, aggr_sh.at[pl.ds(off, ZR)])

    plsc.subcore_barrier()

    base = (c * NS + s) * EPW

    @pl.loop(0, NCHUNK)
    def _chunk(k):
        off = base + k * K
        pltpu.sync_copy(src_hbm.at[pl.ds(off, K)], sidx)
        pltpu.sync_copy(dst_hbm.at[pl.ds(off, K)], didx)
        cpp = pltpu.async_copy(p_hbm.at[sidx], pbuf, sem_p)
        cpq = pltpu.async_copy(q_hbm.at[didx], qbuf, sem_q)
        cpp.wait()
        cpq.wait()

        @pl.loop(0, K)
        def _row(r):
            for j in range(8):
                sl = pl.ds(j * 16, 16)
                pbuf[r, sl] = jnp.maximum(pbuf[r, sl] + qbuf[r, sl], 0.0)

        pltpu.sync_copy(pbuf, aggr_sh.at[didx], add=True)

    plsc.subcore_barrier()
    off = pl.multiple_of(s * RPT, 8)
    pltpu.sync_copy(aggr_sh.at[pl.ds(off, RPT)],
                    out_hbm.at[c, pl.ds(off, RPT)])


@functools.cache
def _get_sc_edge():
    return pl.kernel(
        _sc_edge_body,
        out_type=jax.ShapeDtypeStruct((NC, NP, F), _f32),
        mesh=plsc.VectorSubcoreMesh(core_axis_name="c", subcore_axis_name="s",
                                    num_cores=NC, num_subcores=NS),
        scratch_types=[
            pltpu.VMEM_SHARED((NP, F), _f32),
            pltpu.VMEM((K,), jnp.int32),
            pltpu.VMEM((K,), jnp.int32),
            pltpu.VMEM((K, F), _f32),
            pltpu.VMEM((K, F), _f32),
            pltpu.VMEM((ZR, F), _f32),
            pltpu.SemaphoreType.DMA,
            pltpu.SemaphoreType.DMA,
        ],
    )


def kernel(x, edge_index, batch, W_lin2, b_lin2, We1, be1, W1a, b1a, W1b, b1b,
           W_lin3, b_lin3, We2, be2, W2a, b2a, W2b, b2b, W_lin1, b_lin1):
    src = edge_index[0]
    dst = edge_index[1]
    sds = jax.ShapeDtypeStruct

    p1, q1 = pl.pallas_call(
        _tc_fold1,
        out_shape=[sds((N, F), _f32), sds((N, F), _f32)],
    )(x, W_lin2, b_lin2.reshape(1, H), We1, be1.reshape(1, F))

    sc_edge = _get_sc_edge()
    pa = sc_edge(p1, q1, src, dst)

    p2, q2, xr = pl.pallas_call(
        _tc_mid,
        out_shape=[sds((N, H), _f32), sds((N, H), _f32), sds((N, H), _f32)],
    )(x, pa, W1a, b1a.reshape(1, H), W1b, b1b.reshape(1, H),
      W_lin3, b_lin3.reshape(1, H), We2, be2.reshape(1, H))

    pb = sc_edge(p2, q2, src, dst)

    out = pl.pallas_call(
        _tc_final,
        out_shape=sds((G, C), _f32),
    )(xr, pb, W2a, b2a.reshape(1, H), W2b, b2b.reshape(1, H),
      batch.reshape(N, 1), W_lin1, b_lin1.reshape(1, C))

    return out


# speedup vs baseline: 6.8184x; 1.2296x over previous
import functools

import jax
import jax.numpy as jnp
from jax import lax
from jax.experimental import pallas as pl
from jax.experimental.pallas import tpu as pltpu
from jax.experimental.pallas import tpu_sc as plsc

N = 10000
E = 320000
F = 128
H = 128
C = 10
G = 16

NC = 2
NS = 16
NW = NC * NS
EPW = E // NW
K = 40
NCHUNK = EPW // K
OS = 624
LAST_OS = N - 15 * OS

_f32 = jnp.float32


def _dot(a, b):
    return jnp.dot(a, b, preferred_element_type=_f32)


def _tc_fold1(x_ref, W2_ref, b2_ref, We_ref, be_ref, p_ref, q_ref):
    x = x_ref[...]
    We = We_ref[...]
    A = _dot(W2_ref[0:F, :], We)
    B = _dot(W2_ref[F:2 * F, :], We)
    c = _dot(b2_ref[...], We) + be_ref[...]
    p_ref[...] = x + _dot(x, A)
    q_ref[...] = _dot(x, B) + c


def _bn_mlp(u, Wa, ba, Wb, bb):
    t = _dot(u, Wa) + ba
    mu = jnp.mean(t, axis=0, keepdims=True)
    tc = t - mu
    var = jnp.mean(tc * tc, axis=0, keepdims=True)
    r = jnp.maximum(tc / jnp.sqrt(var + 1e-5), 0.0)
    return _dot(r, Wb) + bb


def _tc_mid(x_ref, pa_ref, W1a_ref, b1a_ref, W1b_ref, b1b_ref,
            W3_ref, b3_ref, We2_ref, be2_ref, p_ref, q_ref, xr_ref):
    u = x_ref[...] + pa_ref[0] + pa_ref[1]
    h = _bn_mlp(u, W1a_ref[...], b1a_ref[...], W1b_ref[...], b1b_ref[...])
    xr = jnp.maximum(h, 0.0)
    We2 = We2_ref[...]
    A = _dot(W3_ref[0:H, :], We2)
    B = _dot(W3_ref[H:2 * H, :], We2)
    c = _dot(b3_ref[...], We2) + be2_ref[...]
    p_ref[...] = xr + _dot(h, A)
    q_ref[...] = _dot(h, B) + c
    xr_ref[...] = xr


def _tc_final(xr_ref, pb_ref, W2a_ref, b2a_ref, W2b_ref, b2b_ref,
              batch_ref, Wl1_ref, bl1_ref, out_ref):
    u = xr_ref[...] + pb_ref[0] + pb_ref[1]
    h2 = _bn_mlp(u, W2a_ref[...], b2a_ref[...], W2b_ref[...], b2b_ref[...])
    hr = jnp.maximum(h2, 0.0)
    onehot = (batch_ref[...] ==
              lax.broadcasted_iota(jnp.int32, (N, G), 1)).astype(_f32)
    pooled = lax.dot_general(onehot, hr, (((0,), (0,)), ((), ())),
                             preferred_element_type=_f32)
    logits = _dot(pooled, Wl1_ref[...]) + bl1_ref[...]
    m = jnp.max(logits, axis=1, keepdims=True)
    lse = jnp.log(jnp.sum(jnp.exp(logits - m), axis=1, keepdims=True)) + m
    out_ref[...] = logits - lse


def _sc_edge_body(p_hbm, q_hbm, src3_hbm, dst3_hbm, out_hbm,
                  aggr_sh, sidx4, didx4,
                  pb0, qb0, mb0, pb1, qb1, mb1,
                  sp0, sq0, ss0, sp1, sq1, ss1,
                  si0, si1, si2, si3, sd0, sd1, sd2, sd3):
    c = lax.axis_index("c")
    s = lax.axis_index("s")
    w = c * NS + s
    bufs = ((pb0, qb0, mb0, sp0, sq0, ss0), (pb1, qb1, mb1, sp1, sq1, ss1))
    sis = (si0, si1, si2, si3)
    sds = (sd0, sd1, sd2, sd3)

    def _idx_issue(kk, slot):
        pltpu.async_copy(src3_hbm.at[w, kk], sidx4.at[slot], sis[slot])
        pltpu.async_copy(dst3_hbm.at[w, kk], didx4.at[slot], sds[slot])

    def _idx_wait(kk, slot):
        pltpu.make_async_copy(src3_hbm.at[w, kk], sidx4.at[slot],
                              sis[slot]).wait()
        pltpu.make_async_copy(dst3_hbm.at[w, kk], didx4.at[slot],
                              sds[slot]).wait()

    def _gather_issue(slot, b):
        pb, qb, _, sp, sq, _ = bufs[b]
        pltpu.async_copy(p_hbm.at[sidx4.at[slot]], pb, sp)
        pltpu.async_copy(q_hbm.at[didx4.at[slot]], qb, sq)

    def _gather_wait(slot, b):
        pb, qb, _, sp, sq, _ = bufs[b]
        pltpu.make_async_copy(p_hbm.at[sidx4.at[slot]], pb, sp).wait()
        pltpu.make_async_copy(q_hbm.at[didx4.at[slot]], qb, sq).wait()

    def _scatter_drain(slot, b):
        mb, ss = bufs[b][2], bufs[b][5]
        pltpu.make_async_copy(mb, aggr_sh.at[didx4.at[slot]], ss).wait()

    def _relu_sum(b):
        pb, qb, mb = bufs[b][0], bufs[b][1], bufs[b][2]

        @pl.loop(0, K)
        def _row(r):
            for j in range(8):
                sl = pl.ds(j * 16, 16)
                mb[r, sl] = jnp.maximum(pb[r, sl] + qb[r, sl], 0.0)

    _idx_issue(0, 0)
    _idx_issue(1, 1)

    @pl.loop(0, 8)
    def _zrow(r):
        for j in range(8):
            mb0[r, pl.ds(j * 16, 16)] = jnp.zeros((16,), _f32)

    nz = jnp.where(s == NS - 1, LAST_OS // 8, OS // 8)

    @pl.loop(0, nz)
    def _zcopy(j):
        off = pl.multiple_of(s * OS + j * 8, 8)
        pltpu.sync_copy(mb0.at[pl.ds(0, 8)], aggr_sh.at[pl.ds(off, 8)])

    plsc.subcore_barrier()

    _idx_wait(0, 0)
    _gather_issue(0, 0)
    _idx_issue(2, 2)

    @pl.loop(0, NCHUNK - 2, step=4)
    def _main(k):
        for b in range(4):
            b2 = b % 2
            kk = k + b
            _gather_wait(b, b2)
            if b < 2:
                @pl.when(k > 0)
                def _drain():
                    _scatter_drain((b + 2) % 4, b2)
            else:
                _scatter_drain((b + 2) % 4, b2)
            _relu_sum(b2)
            mb, ss = bufs[b2][2], bufs[b2][5]
            pltpu.async_copy(mb, aggr_sh.at[didx4.at[b]], ss, add=True)
            _idx_wait(kk + 1, (b + 1) % 4)
            _gather_issue((b + 1) % 4, 1 - b2)
            _idx_issue(kk + 2, (b + 2) % 4)

    _gather_wait(0, 0)
    _scatter_drain(2, 0)
    _relu_sum(0)
    pltpu.async_copy(mb0, aggr_sh.at[didx4.at[0]], ss0, add=True)
    _idx_wait(NCHUNK - 1, 1)
    _gather_issue(1, 1)
    _gather_wait(1, 1)
    _scatter_drain(3, 1)
    _relu_sum(1)
    pltpu.async_copy(mb1, aggr_sh.at[didx4.at[1]], ss1, add=True)
    _scatter_drain(0, 0)
    _scatter_drain(1, 1)

    plsc.subcore_barrier()
    off = pl.multiple_of(s * OS, 8)

    @pl.when(s < NS - 1)
    def _copy_out():
        pltpu.sync_copy(aggr_sh.at[pl.ds(off, OS)],
                        out_hbm.at[c, pl.ds(off, OS)])

    @pl.when(s == NS - 1)
    def _copy_out_last():
        pltpu.sync_copy(aggr_sh.at[pl.ds(off, LAST_OS)],
                        out_hbm.at[c, pl.ds(off, LAST_OS)])


@functools.cache
def _get_sc_edge():
    return pl.kernel(
        _sc_edge_body,
        out_type=jax.ShapeDtypeStruct((NC, N, F), _f32),
        mesh=plsc.VectorSubcoreMesh(core_axis_name="c", subcore_axis_name="s",
                                    num_cores=NC, num_subcores=NS),
        scratch_types=(
            [pltpu.VMEM_SHARED((N, F), _f32)]
            + [pltpu.VMEM((4, K), jnp.int32)] * 2
            + [pltpu.VMEM((K, F), _f32)] * 6
            + [pltpu.SemaphoreType.DMA] * 14
        ),
    )


def kernel(x, edge_index, batch, W_lin2, b_lin2, We1, be1, W1a, b1a, W1b, b1b,
           W_lin3, b_lin3, We2, be2, W2a, b2a, W2b, b2b, W_lin1, b_lin1):
    src = edge_index[0].reshape(NW, NCHUNK, K)
    dst = edge_index[1].reshape(NW, NCHUNK, K)
    sds = jax.ShapeDtypeStruct

    p1, q1 = pl.pallas_call(
        _tc_fold1,
        out_shape=[sds((N, F), _f32), sds((N, F), _f32)],
    )(x, W_lin2, b_lin2.reshape(1, H), We1, be1.reshape(1, F))

    sc_edge = _get_sc_edge()
    pa = sc_edge(p1, q1, src, dst)

    p2, q2, xr = pl.pallas_call(
        _tc_mid,
        out_shape=[sds((N, H), _f32), sds((N, H), _f32), sds((N, H), _f32)],
    )(x, pa, W1a, b1a.reshape(1, H), W1b, b1b.reshape(1, H),
      W_lin3, b_lin3.reshape(1, H), We2, be2.reshape(1, H))

    pb = sc_edge(p2, q2, src, dst)

    out = pl.pallas_call(
        _tc_final,
        out_shape=sds((G, C), _f32),
    )(xr, pb, W2a, b2a.reshape(1, H), W2b, b2b.reshape(1, H),
      batch.reshape(N, 1), W_lin1, b_lin1.reshape(1, C))

    return out
